# baseline (speedup 1.0000x reference)
import functools

import jax
import jax.numpy as jnp
from jax import lax
from jax.experimental import pallas as pl
from jax.experimental.pallas import tpu as pltpu
from jax.experimental.pallas import tpu_sc as plsc

_NC = 2
_NS = 16
_NW = _NC * _NS
_CH = 96
_L = 16
_BC = 8
_IB = _BC * _CH
_ZR = 40


def _acc_rows(n):
    unit = _NS * _ZR
    return (n + 1 + unit - 1) // unit * unit


@functools.lru_cache(maxsize=None)
def _make_agg(n, w, e_pad):
    na = _acc_rows(n)
    zr = na // _NS
    ept = e_pad // _NW
    t_steps = ept // _CH
    pairs = t_steps // 2
    blocks_per_tile = ept // _IB
    mesh = plsc.VectorSubcoreMesh(core_axis_name="c", subcore_axis_name="s")

    @functools.partial(
        pl.kernel,
        out_type=jax.ShapeDtypeStruct((_NC * na, w), jnp.float32),
        mesh=mesh,
        scratch_types=[
            pltpu.VMEM((_BC, _CH), jnp.int32),
            pltpu.VMEM((_BC, _CH), jnp.int32),
            pltpu.VMEM((_CH,), jnp.int32),
            pltpu.VMEM((_CH,), jnp.int32),
            pltpu.VMEM((_CH,), jnp.int32),
            pltpu.VMEM((_CH,), jnp.int32),
            pltpu.VMEM((_CH, w), jnp.float32),
            pltpu.VMEM((_CH, w), jnp.float32),
            pltpu.VMEM((_ZR, w), jnp.float32),
            pltpu.VMEM_SHARED((na, w), jnp.float32),
            pltpu.SemaphoreType.DMA,
            pltpu.SemaphoreType.DMA,
            pltpu.SemaphoreType.DMA,
            pltpu.SemaphoreType.DMA,
        ],
        compiler_params=pltpu.CompilerParams(use_tc_tiling_on_sc=False),
    )
    def agg(xp_hbm, src_hbm, dst_hbm, out_hbm, sblk, dblk, srcb0, srcb1,
            dstb0, dstb1, rows0, rows1, zer_v, acc_sh,
            semg0, semg1, sems0, sems1):
        c = lax.axis_index("c")
        s = lax.axis_index("s")
        wid = c * _NS + s

        zvec = jnp.zeros((_L,), jnp.float32)
        cpr = w // _L

        def zbody(i, carry):
            zer_v[i // cpr, pl.ds((i % cpr) * _L, _L)] = zvec
            return carry

        lax.fori_loop(0, _ZR * cpr, zbody, 0)
        for i in range(zr // _ZR):
            pltpu.sync_copy(zer_v, acc_sh.at[pl.ds(s * zr + i * _ZR, _ZR)])
        plsc.subcore_barrier()

        def load_block(g):
            gb = wid * blocks_per_tile + g
            pltpu.sync_copy(src_hbm.at[gb], sblk)
            pltpu.sync_copy(dst_hbm.at[gb], dblk)

        def prep(t, srcb, dstb):
            row = t % _BC

            def kbody(k, carry2):
                sl = pl.ds(k * _L, _L)
                srcb[sl] = sblk[row, sl]
                dstb[sl] = dblk[row, sl]
                return carry2

            lax.fori_loop(0, _CH // _L, kbody, 0)

        def start_gather(srcb, rows, semg):
            pltpu.async_copy(xp_hbm.at[srcb], rows, semg)

        def wait_gather(srcb, rows, semg):
            pltpu.make_async_copy(xp_hbm.at[srcb], rows, semg).wait()

        def start_scatter(rows, dstb, sems):
            pltpu.async_copy(rows, acc_sh.at[dstb], sems, add=True)

        def wait_scatter(rows, dstb, sems):
            pltpu.make_async_copy(rows, acc_sh.at[dstb], sems).wait()

        load_block(0)
        prep(0, srcb0, dstb0)
        start_gather(srcb0, rows0, semg0)

        def pbody(p, carry):
            t0 = 2 * p

            @pl.when(p > 0)
            def _():
                wait_scatter(rows1, dstb1, sems1)

            prep(t0 + 1, srcb1, dstb1)
            start_gather(srcb1, rows1, semg1)
            wait_gather(srcb0, rows0, semg0)
            start_scatter(rows0, dstb0, sems0)

            wait_scatter(rows0, dstb0, sems0)

            @pl.when((p % (_BC // 2)) == (_BC // 2 - 1))
            def _():
                g = (2 * p + 2) // _BC

                @pl.when(g < blocks_per_tile)
                def _():
                    load_block(g)

            @pl.when(p < pairs - 1)
            def _():
                prep(t0 + 2, srcb0, dstb0)
                start_gather(srcb0, rows0, semg0)

            wait_gather(srcb1, rows1, semg1)
            start_scatter(rows1, dstb1, sems1)
            return carry

        lax.fori_loop(0, pairs, pbody, 0)
        wait_scatter(rows1, dstb1, sems1)
        plsc.subcore_barrier()

        pltpu.sync_copy(acc_sh.at[pl.ds(s * zr, zr)],
                        out_hbm.at[pl.ds(c * na + s * zr, zr)])

    return agg


@functools.lru_cache(maxsize=None)
def _make_mlp(n, w):
    na = _acc_rows(n)
    d = 128

    def body(parts_ref, x_ref, w1_ref, b1_ref, g_ref, be_ref, w2_ref, b2_ref,
             fl_ref, out_ref):
        agg = parts_ref[0:n, :] + parts_ref[na:na + n, :]
        ssum = agg[:, 0:d]
        cnt = jnp.sum(agg[:, d:w], axis=1, keepdims=True)
        mean = ssum / jnp.maximum(cnt, 1.0)
        h = x_ref[:, 0:d] + mean
        z = lax.dot_general(h, w1_ref[...], (((1,), (1,)), ((), ())),
                            preferred_element_type=jnp.float32)
        z = z + b1_ref[...][None, :]
        mu = jnp.mean(z, axis=0, keepdims=True)
        zc = z - mu
        var = jnp.mean(zc * zc, axis=0, keepdims=True)
        zn = zc * (g_ref[...][None, :] * lax.rsqrt(var + 1e-5))
        zn = zn + be_ref[...][None, :]
        act = jnp.maximum(zn, 0.0)
        o = lax.dot_general(act, w2_ref[...], (((1,), (1,)), ((), ())),
                            preferred_element_type=jnp.float32)
        o = o + b2_ref[...][None, :]
        o = jnp.where(fl_ref[...] > 0.0, jnp.maximum(o, 0.0), o)
        lane = lax.broadcasted_iota(jnp.int32, (n, w - d), 1)
        pad = jnp.where(lane == 0, 1.0, 0.0)
        out_ref[...] = jnp.concatenate([o, pad], axis=1)

    return pl.pallas_call(
        body,
        out_shape=jax.ShapeDtypeStruct((n, w), jnp.float32),
    )


def kernel(x, edge_index, W1_0, b1_0, g_0, be_0, W2_0, b2_0,
           W1_1, b1_1, g_1, be_1, W2_1, b2_1):
    n, d = x.shape
    e = edge_index.shape[1]
    w = d + 16
    ept = ((e + _NW - 1) // _NW + _IB - 1) // _IB * _IB
    e_pad = ept * _NW
    pad_e = e_pad - e

    src_p = jnp.concatenate(
        [edge_index[0].astype(jnp.int32), jnp.zeros((pad_e,), jnp.int32)])
    dst_p = jnp.concatenate(
        [edge_index[1].astype(jnp.int32), jnp.full((pad_e,), n, jnp.int32)])
    src_p = src_p.reshape(-1, _BC, _CH)
    dst_p = dst_p.reshape(-1, _BC, _CH)
    onescol = jnp.concatenate(
        [jnp.ones((n, 1), jnp.float32), jnp.zeros((n, w - d - 1), jnp.float32)],
        axis=1)
    xp = jnp.concatenate([x.astype(jnp.float32), onescol], axis=1)

    agg = _make_agg(n, w, e_pad)
    mlp = _make_mlp(n, w)

    w1s = jnp.stack([W1_0, W1_1])
    b1s = jnp.stack([b1_0, b1_1])
    gs = jnp.stack([g_0, g_1])
    bes = jnp.stack([be_0, be_1])
    w2s = jnp.stack([W2_0, W2_1])
    b2s = jnp.stack([b2_0, b2_1])
    flags = jnp.stack([jnp.ones((1, d), jnp.float32),
                       jnp.zeros((1, d), jnp.float32)])

    def step(carry, xs):
        w1, b1, g, be, w2, b2, fl = xs
        parts = agg(carry, src_p, dst_p)
        h = mlp(parts, carry, w1, b1, g, be, w2, b2, fl)
        return h, None

    hfinal, _ = lax.scan(step, xp, (w1s, b1s, gs, bes, w2s, b2s, flags))
    return hfinal[:, 0:d]


# speedup vs baseline: 7.2686x; 7.2686x over previous
import functools

import jax
import jax.numpy as jnp
from jax import lax
from jax.experimental import pallas as pl
from jax.experimental.pallas import tpu as pltpu
from jax.experimental.pallas import tpu_sc as plsc

_NC = 2
_NS = 16
_NW = _NC * _NS
_CH = 128
_L = 16
_BC = 8
_IB = _BC * _CH
_ZR = 40


def _acc_rows(n):
    unit = _NS * _ZR
    return (n + 1 + unit - 1) // unit * unit


@functools.lru_cache(maxsize=None)
def _make_agg(n, d, e_pad):
    na = _acc_rows(n)
    zr = na // _NS
    ept = e_pad // _NW
    t_steps = ept // _CH
    pairs = t_steps // 2
    blocks_per_tile = ept // _IB
    mesh = plsc.VectorSubcoreMesh(core_axis_name="c", subcore_axis_name="s")

    @functools.partial(
        pl.kernel,
        out_type=jax.ShapeDtypeStruct((_NC * na, d), jnp.float32),
        mesh=mesh,
        scratch_types=[
            pltpu.VMEM((_BC, _CH), jnp.int32),
            pltpu.VMEM((_BC, _CH), jnp.int32),
            pltpu.VMEM((_CH,), jnp.int32),
            pltpu.VMEM((_CH,), jnp.int32),
            pltpu.VMEM((_CH,), jnp.int32),
            pltpu.VMEM((_CH,), jnp.int32),
            pltpu.VMEM((_CH, d), jnp.float32),
            pltpu.VMEM((_CH, d), jnp.float32),
            pltpu.VMEM((_ZR, d), jnp.float32),
            pltpu.VMEM_SHARED((na, d), jnp.float32),
            pltpu.SemaphoreType.DMA,
            pltpu.SemaphoreType.DMA,
            pltpu.SemaphoreType.DMA,
            pltpu.SemaphoreType.DMA,
        ],
        compiler_params=pltpu.CompilerParams(needs_layout_passes=False),
    )
    def agg(x_hbm, src_hbm, dst_hbm, out_hbm, sblk, dblk, srcb0, srcb1,
            dstb0, dstb1, rows0, rows1, zer_v, acc_sh,
            semg0, semg1, sems0, sems1):
        c = lax.axis_index("c")
        s = lax.axis_index("s")
        wid = c * _NS + s

        zvec = jnp.zeros((_L,), jnp.float32)
        cpr = d // _L

        def zbody(i, carry):
            zer_v[i // cpr, pl.ds((i % cpr) * _L, _L)] = zvec
            return carry

        lax.fori_loop(0, _ZR * cpr, zbody, 0)
        for i in range(zr // _ZR):
            pltpu.sync_copy(zer_v, acc_sh.at[pl.ds(s * zr + i * _ZR, _ZR)])
        plsc.subcore_barrier()

        def load_block(g):
            gb = wid * blocks_per_tile + g
            pltpu.sync_copy(src_hbm.at[gb], sblk)
            pltpu.sync_copy(dst_hbm.at[gb], dblk)

        def prep(t, srcb, dstb):
            row = t % _BC

            def kbody(k, carry2):
                sl = pl.ds(k * _L, _L)
                srcb[sl] = sblk[row, sl]
                dstb[sl] = dblk[row, sl]
                return carry2

            lax.fori_loop(0, _CH // _L, kbody, 0)

        def start_gather(srcb, rows, semg):
            pltpu.async_copy(x_hbm.at[srcb], rows, semg)

        def wait_gather(srcb, rows, semg):
            pltpu.make_async_copy(x_hbm.at[srcb], rows, semg).wait()

        def start_scatter(rows, dstb, sems):
            pltpu.async_copy(rows, acc_sh.at[dstb], sems, add=True)

        def wait_scatter(rows, dstb, sems):
            pltpu.make_async_copy(rows, acc_sh.at[dstb], sems).wait()

        load_block(0)
        prep(0, srcb0, dstb0)
        start_gather(srcb0, rows0, semg0)

        def pbody(p, carry):
            t0 = 2 * p

            @pl.when(p > 0)
            def _():
                wait_scatter(rows1, dstb1, sems1)

            prep(t0 + 1, srcb1, dstb1)
            start_gather(srcb1, rows1, semg1)
            wait_gather(srcb0, rows0, semg0)
            start_scatter(rows0, dstb0, sems0)

            wait_scatter(rows0, dstb0, sems0)

            @pl.when((p % (_BC // 2)) == (_BC // 2 - 1))
            def _():
                g = (2 * p + 2) // _BC

                @pl.when(g < blocks_per_tile)
                def _():
                    load_block(g)

            @pl.when(p < pairs - 1)
            def _():
                prep(t0 + 2, srcb0, dstb0)
                start_gather(srcb0, rows0, semg0)

            wait_gather(srcb1, rows1, semg1)
            start_scatter(rows1, dstb1, sems1)
            return carry

        lax.fori_loop(0, pairs, pbody, 0)
        wait_scatter(rows1, dstb1, sems1)
        plsc.subcore_barrier()

        pltpu.sync_copy(acc_sh.at[pl.ds(s * zr, zr)],
                        out_hbm.at[pl.ds(c * na + s * zr, zr)])

    return agg


@functools.lru_cache(maxsize=None)
def _make_counts(n, e_pad):
    na = _acc_rows(n)
    ept = e_pad // _NW
    blocks_per_tile = ept // _IB
    mesh = plsc.VectorSubcoreMesh(core_axis_name="c", subcore_axis_name="s")

    @functools.partial(
        pl.kernel,
        out_type=jax.ShapeDtypeStruct((_NW, na), jnp.float32),
        mesh=mesh,
        scratch_types=[
            pltpu.VMEM((_BC, _CH), jnp.int32),
            pltpu.VMEM((na,), jnp.float32),
        ],
        compiler_params=pltpu.CompilerParams(needs_layout_passes=False),
    )
    def counts(dst_hbm, out_hbm, dblk, cnt_v):
        c = lax.axis_index("c")
        s = lax.axis_index("s")
        wid = c * _NS + s

        zvec = jnp.zeros((_L,), jnp.float32)

        def zbody(i, carry):
            cnt_v[pl.ds(i * _L, _L)] = zvec
            return carry

        lax.fori_loop(0, na // _L, zbody, 0)

        ones = jnp.ones((_L,), jnp.float32)
        steps_per_block = _IB // _L

        def gbody(g, carry):
            pltpu.sync_copy(dst_hbm.at[wid * blocks_per_tile + g], dblk)

            def ibody(i, carry2):
                row = i // (_CH // _L)
                k = i % (_CH // _L)
                v = dblk[row, pl.ds(k * _L, _L)]
                plsc.addupdate_scatter(cnt_v, [v], ones)
                return carry2

            lax.fori_loop(0, steps_per_block, ibody, 0)
            return carry

        lax.fori_loop(0, blocks_per_tile, gbody, 0)
        pltpu.sync_copy(cnt_v, out_hbm.at[wid])

    return counts


@functools.lru_cache(maxsize=None)
def _make_mlp(n, d):
    na = _acc_rows(n)

    def body(parts_ref, x_ref, cnt_ref, w1_ref, b1_ref, g_ref, be_ref,
             w2_ref, b2_ref, fl_ref, out_ref):
        agg = parts_ref[0:n, :] + parts_ref[na:na + n, :]
        cnt = jnp.max(cnt_ref[...], axis=1, keepdims=True)
        mean = agg / jnp.maximum(cnt, 1.0)
        h = x_ref[...] + mean
        z = lax.dot_general(h, w1_ref[...], (((1,), (1,)), ((), ())),
                            preferred_element_type=jnp.float32)
        z = z + b1_ref[...][None, :]
        mu = jnp.mean(z, axis=0, keepdims=True)
        zc = z - mu
        var = jnp.mean(zc * zc, axis=0, keepdims=True)
        zn = zc * (g_ref[...][None, :] * lax.rsqrt(var + 1e-5))
        zn = zn + be_ref[...][None, :]
        act = jnp.maximum(zn, 0.0)
        o = lax.dot_general(act, w2_ref[...], (((1,), (1,)), ((), ())),
                            preferred_element_type=jnp.float32)
        o = o + b2_ref[...][None, :]
        out_ref[...] = jnp.where(fl_ref[...] > 0.0, jnp.maximum(o, 0.0), o)

    return pl.pallas_call(
        body,
        out_shape=jax.ShapeDtypeStruct((n, d), jnp.float32),
    )


def kernel(x, edge_index, W1_0, b1_0, g_0, be_0, W2_0, b2_0,
           W1_1, b1_1, g_1, be_1, W2_1, b2_1):
    n, d = x.shape
    e = edge_index.shape[1]
    na = _acc_rows(n)
    ept = ((e + _NW - 1) // _NW + _IB - 1) // _IB * _IB
    e_pad = ept * _NW
    pad_e = e_pad - e

    pad_i = jnp.arange(pad_e, dtype=jnp.int32)
    src_p = jnp.concatenate(
        [edge_index[0].astype(jnp.int32), pad_i % jnp.int32(n)])
    dst_p = jnp.concatenate(
        [edge_index[1].astype(jnp.int32), n + pad_i % jnp.int32(na - n)])
    src_p = src_p.reshape(-1, _BC, _CH)
    dst_p = dst_p.reshape(-1, _BC, _CH)
    xf = x.astype(jnp.float32)

    agg = _make_agg(n, d, e_pad)
    mlp = _make_mlp(n, d)

    cnts = _make_counts(n, e_pad)(dst_p)
    cnt8 = jnp.broadcast_to(jnp.sum(cnts, axis=0)[:n, None], (n, 8))

    w1s = jnp.stack([W1_0, W1_1])
    b1s = jnp.stack([b1_0, b1_1])
    gs = jnp.stack([g_0, g_1])
    bes = jnp.stack([be_0, be_1])
    w2s = jnp.stack([W2_0, W2_1])
    b2s = jnp.stack([b2_0, b2_1])
    flags = jnp.stack([jnp.ones((1, d), jnp.float32),
                       jnp.zeros((1, d), jnp.float32)])

    def step(carry, xs):
        w1, b1, g, be, w2, b2, fl = xs
        parts = agg(carry, src_p, dst_p)
        h = mlp(parts, carry, cnt8, w1, b1, g, be, w2, b2, fl)
        return h, None

    hfinal, _ = lax.scan(step, xf, (w1s, b1s, gs, bes, w2s, b2s, flags))
    return hfinal
